# baseline (speedup 1.0000x reference)
import jax
import jax.numpy as jnp
from jax.experimental import pallas as pl
from jax.experimental.pallas import tpu as pltpu

_IDX0 = 7
_STRIDE = 156
_NPAIRS = 64
_NUM_NODES = 10000
_W = 2 * _NUM_NODES

_BLOCK_ROWS = 32
_NBUF = 8


def _body(x_ref, o_ref, scr, sem):
    i = pl.program_id(0)
    n = pl.num_programs(0)
    r = _BLOCK_ROWS
    b = jax.lax.rem(i, _NBUF)

    @pl.when(i == 0)
    def _():
        scr[...] = jnp.zeros_like(scr)

    @pl.when(i >= _NBUF)
    def _():
        pltpu.make_async_copy(
            scr.at[b], o_ref.at[pl.ds((i - _NBUF) * r, r), :], sem.at[b]
        ).wait()

    x = x_ref[...]
    for k in range(_NPAIRS):
        col = 2 * (_IDX0 + _STRIDE * k)
        scr[b, :, col:col + 2] = x[:, 2 * k:2 * k + 2]

    pltpu.make_async_copy(
        scr.at[b], o_ref.at[pl.ds(i * r, r), :], sem.at[b]
    ).start()

    @pl.when(i == n - 1)
    def _():
        for j in range(_NBUF):
            s = n - _NBUF + j
            pltpu.make_async_copy(
                scr.at[s % _NBUF], o_ref.at[pl.ds(s * r, r), :],
                sem.at[s % _NBUF]
            ).wait()


def kernel(f_lat):
    rows = f_lat.shape[0]
    out = pl.pallas_call(
        _body,
        grid=(rows // _BLOCK_ROWS,),
        in_specs=[pl.BlockSpec((_BLOCK_ROWS, 128), lambda i: (i, 0))],
        out_specs=pl.BlockSpec(memory_space=pl.ANY),
        out_shape=jax.ShapeDtypeStruct((rows, _W), f_lat.dtype),
        scratch_shapes=[
            pltpu.VMEM((_NBUF, _BLOCK_ROWS, _W), jnp.float32),
            pltpu.SemaphoreType.DMA((_NBUF,)),
        ],
    )(f_lat)
    return out.reshape(rows, _NUM_NODES, 2)


# speedup vs baseline: 1.0034x; 1.0034x over previous
import jax
import jax.numpy as jnp
from jax.experimental import pallas as pl
from jax.experimental.pallas import tpu as pltpu

_IDX0 = 7
_STRIDE = 156
_NPAIRS = 64
_NUM_NODES = 10000
_W = 2 * _NUM_NODES
_BLOCK_ROWS = 64


def _body(x_ref, o_ref):
    o_ref[...] = jnp.zeros_like(o_ref)
    x = x_ref[...]
    for k in range(_NPAIRS):
        col = 2 * (_IDX0 + _STRIDE * k)
        o_ref[:, col:col + 2] = x[:, 2 * k:2 * k + 2]


def kernel(f_lat):
    rows = f_lat.shape[0]
    out = pl.pallas_call(
        _body,
        grid=(rows // _BLOCK_ROWS,),
        in_specs=[pl.BlockSpec((_BLOCK_ROWS, 128), lambda i: (i, 0))],
        out_specs=pl.BlockSpec((_BLOCK_ROWS, _W), lambda i: (i, 0)),
        out_shape=jax.ShapeDtypeStruct((rows, _W), f_lat.dtype),
        compiler_params=pltpu.CompilerParams(
            dimension_semantics=("parallel",)),
    )(f_lat)
    return out.reshape(rows, _NUM_NODES, 2)


# speedup vs baseline: 6.7854x; 6.7623x over previous
import jax
import jax.numpy as jnp
from jax.experimental import pallas as pl
from jax.experimental.pallas import tpu as pltpu

_IDX0 = 7
_STRIDE = 156
_NPAIRS = 64
_NUM_NODES = 10000
_TAIL = _NUM_NODES - _NPAIRS * _STRIDE
_NBUF = 4


def _body(e_ref, o_ref, scr, ztail, sem, zsem):
    i = pl.program_id(0)
    n = pl.num_programs(0)
    b = jax.lax.rem(i, _NBUF)

    @pl.when(i == 0)
    def _():
        scr[...] = jnp.zeros_like(scr)
        ztail[...] = jnp.zeros_like(ztail)

    @pl.when(i < _NPAIRS)
    def _():
        @pl.when(i >= _NBUF)
        def _():
            pltpu.make_async_copy(
                scr.at[b], o_ref.at[pl.ds((i - _NBUF) * _STRIDE, _STRIDE)],
                sem.at[b]).wait()

        scr[b, _IDX0] = e_ref[i]
        pltpu.make_async_copy(
            scr.at[b], o_ref.at[pl.ds(i * _STRIDE, _STRIDE)],
            sem.at[b]).start()

    @pl.when(i == n - 1)
    def _():
        pltpu.make_async_copy(
            ztail, o_ref.at[pl.ds(_NPAIRS * _STRIDE, _TAIL)], zsem).start()
        for j in range(_NBUF):
            s = _NPAIRS - _NBUF + j
            pltpu.make_async_copy(
                scr.at[s % _NBUF], o_ref.at[pl.ds(s * _STRIDE, _STRIDE)],
                sem.at[s % _NBUF]).wait()
        pltpu.make_async_copy(
            ztail, o_ref.at[pl.ds(_NPAIRS * _STRIDE, _TAIL)], zsem).wait()


def kernel(f_lat):
    rows = f_lat.shape[0]
    groups = rows // 128

    e = (
        f_lat.reshape(groups, 128, _NPAIRS, 2)
        .transpose(2, 0, 3, 1)
        .reshape(_NPAIRS, 2 * groups, 128)
    )

    out = pl.pallas_call(
        _body,
        grid=(_NPAIRS + 1,),
        in_specs=[pl.BlockSpec((_NPAIRS, 2 * groups, 128), lambda i: (0, 0, 0))],
        out_specs=pl.BlockSpec(memory_space=pl.ANY),
        out_shape=jax.ShapeDtypeStruct((_NUM_NODES, 2 * groups, 128),
                                       f_lat.dtype),
        scratch_shapes=[
            pltpu.VMEM((_NBUF, _STRIDE, 2 * groups, 128), jnp.float32),
            pltpu.VMEM((_TAIL, 2 * groups, 128), jnp.float32),
            pltpu.SemaphoreType.DMA((_NBUF,)),
            pltpu.SemaphoreType.DMA,
        ],
    )(e)

    return (
        out.reshape(_NUM_NODES, groups, 2, 128)
        .transpose(1, 3, 0, 2)
        .reshape(rows, _NUM_NODES, 2)
    )
